# Initial kernel scaffold; baseline (speedup 1.0000x reference)
import jax
import jax.numpy as jnp
from jax.experimental import pallas as pl


def kernel(input_ids, weight, comp_map):
    raise NotImplementedError("write your pallas kernel here")


# speedup vs baseline: 3.7876x; 3.7876x over previous
import functools

import jax
import jax.numpy as jnp
from jax import lax
from jax.experimental import pallas as pl
from jax.experimental.pallas import tpu as pltpu
from jax.experimental.pallas import tpu_sc as plsc

VOCAB = 16
D = 1024
TOKENS = 4 * 8192

_info = plsc.get_sparse_core_info()
NC, NS = _info.num_cores, _info.num_subcores
NW = NC * NS
TPW = TOKENS // NW
CHUNK = 16
NCHUNK = TPW // CHUNK


def _build_trc_body(w_ref, cm_ref, trc_ref):
    w = w_ref[...]
    cm = cm_ref[...]
    onehot = (cm == lax.broadcasted_iota(jnp.int32, (VOCAB, VOCAB), 1))
    sel = jax.lax.dot(onehot.astype(jnp.float32), w,
                      precision=jax.lax.Precision.HIGHEST)
    revp = (lax.broadcasted_iota(jnp.int32, (D, D), 0)
            + lax.broadcasted_iota(jnp.int32, (D, D), 1)) == (D - 1)
    trc_ref[...] = jax.lax.dot(sel, revp.astype(jnp.float32),
                               precision=jax.lax.Precision.HIGHEST)


def _build_trc(weight, comp_map):
    return pl.pallas_call(
        _build_trc_body,
        out_shape=jax.ShapeDtypeStruct((VOCAB, D), jnp.float32),
    )(weight, comp_map.reshape(VOCAB, 1))


def _sc_gather_body(ids_hbm, w_hbm, trc_hbm, out_hbm,
                    ids_v, fwd_v, rc_v, semg):
    wid = lax.axis_index("s") * NC + lax.axis_index("c")
    base = wid * TPW
    pltpu.sync_copy(ids_hbm.at[pl.ds(base, TPW)], ids_v)

    def chunk(i, carry):
        off = i * CHUNK
        idx = ids_v.at[pl.ds(off, CHUNK)]
        cp1 = pltpu.async_copy(w_hbm.at[idx], fwd_v, semg)
        cp2 = pltpu.async_copy(trc_hbm.at[idx], rc_v, semg)
        cp1.wait()
        cp2.wait()
        pltpu.sync_copy(fwd_v, out_hbm.at[pl.ds(base + off, CHUNK), pl.ds(0, D)])
        pltpu.sync_copy(rc_v, out_hbm.at[pl.ds(base + off, CHUNK), pl.ds(D, D)])
        return carry

    lax.fori_loop(0, NCHUNK, chunk, 0)


@functools.partial(jax.jit, static_argnames=())
def _sc_gather(ids, weight, trc):
    mesh = plsc.VectorSubcoreMesh(core_axis_name="c", subcore_axis_name="s")
    f = functools.partial(
        pl.kernel,
        mesh=mesh,
        out_type=jax.ShapeDtypeStruct((TOKENS, 2 * D), jnp.float32),
        scratch_types=[
            pltpu.VMEM((TPW,), jnp.int32),
            pltpu.VMEM((CHUNK, D), jnp.float32),
            pltpu.VMEM((CHUNK, D), jnp.float32),
            pltpu.SemaphoreType.DMA,
        ],
    )(_sc_gather_body)
    return f(ids, weight, trc)


def kernel(input_ids, weight, comp_map):
    ids = input_ids.reshape(-1)
    trc = _build_trc(weight, comp_map)
    out = _sc_gather(ids, weight, trc)
    return out.reshape(input_ids.shape[0], input_ids.shape[1], 2 * D)
